# Initial kernel scaffold; baseline (speedup 1.0000x reference)
import jax
import jax.numpy as jnp
from jax.experimental import pallas as pl


def kernel(src_table, tgt_table, src_indices, tgt_indices):
    raise NotImplementedError("write your pallas kernel here")


# speedup vs baseline: 4.3127x; 4.3127x over previous
import functools

import jax
import jax.numpy as jnp
from jax import lax
from jax.experimental import pallas as pl
from jax.experimental.pallas import tpu as pltpu
from jax.experimental.pallas import tpu_sc as plsc

NC, NS = 2, 16
NW = NC * NS
CHUNK = 128


@functools.partial(jax.jit, static_argnums=(4, 5))
def _dual_gather(src_table, tgt_table, src_idx, tgt_idx, n, d):
    n_w = n // NW
    n_chunks = n_w // CHUNK
    mesh = plsc.VectorSubcoreMesh(core_axis_name="c", subcore_axis_name="s")

    @functools.partial(
        pl.kernel,
        out_type=(
            jax.ShapeDtypeStruct((n, d), jnp.float32),
            jax.ShapeDtypeStruct((n, d), jnp.float32),
        ),
        mesh=mesh,
        scratch_types=[
            pltpu.VMEM((n_chunks, CHUNK), jnp.int32),
            pltpu.VMEM((n_chunks, CHUNK), jnp.int32),
            pltpu.VMEM((CHUNK, d), jnp.float32),
            pltpu.SemaphoreType.DMA,
        ],
        compiler_params=pltpu.CompilerParams(use_tc_tiling_on_sc=False),
    )
    def k(src_tab, tgt_tab, sidx_hbm, tidx_hbm, src_out, tgt_out,
          sidx_v, tidx_v, rows_v, sem):
        wid = lax.axis_index("s") * NC + lax.axis_index("c")
        base = wid * n_w
        pltpu.sync_copy(sidx_hbm.at[wid], sidx_v)
        pltpu.sync_copy(tidx_hbm.at[wid], tidx_v)

        def body(j, _):
            pltpu.async_copy(src_tab.at[sidx_v.at[j]], rows_v, sem).wait()
            pltpu.sync_copy(rows_v, src_out.at[pl.ds(base + j * CHUNK, CHUNK)])
            pltpu.async_copy(tgt_tab.at[tidx_v.at[j]], rows_v, sem).wait()
            pltpu.sync_copy(rows_v, tgt_out.at[pl.ds(base + j * CHUNK, CHUNK)])
            return 0

        lax.fori_loop(0, n_chunks, body, 0)

    return k(src_table, tgt_table, src_idx, tgt_idx)


def kernel(src_table, tgt_table, src_indices, tgt_indices):
    b, l = src_indices.shape
    d = src_table.shape[1]
    n = b * l
    n_w = n // NW
    n_chunks = n_w // CHUNK
    sidx = src_indices.astype(jnp.int32).reshape(NW, n_chunks, CHUNK)
    tidx = tgt_indices.astype(jnp.int32).reshape(NW, n_chunks, CHUNK)
    src_flat, tgt_flat = _dual_gather(src_table, tgt_table, sidx, tidx, n, d)
    return (src_flat.reshape(b, l, d), tgt_flat.reshape(b, l, d))


# speedup vs baseline: 4.9484x; 1.1474x over previous
import functools

import jax
import jax.numpy as jnp
from jax import lax
from jax.experimental import pallas as pl
from jax.experimental.pallas import tpu as pltpu
from jax.experimental.pallas import tpu_sc as plsc

NC, NS = 2, 16
NW = NC * NS
CHUNK = 128
K = 5


@functools.partial(jax.jit, static_argnums=(4, 5))
def _dual_gather(src_table, tgt_table, src_idx, tgt_idx, n, d):
    n_w = n // NW
    n_chunks = n_w // CHUNK
    n_pairs = n_chunks // K
    mesh = plsc.VectorSubcoreMesh(core_axis_name="c", subcore_axis_name="s")

    @functools.partial(
        pl.kernel,
        out_type=(
            jax.ShapeDtypeStruct((n, d), jnp.float32),
            jax.ShapeDtypeStruct((n, d), jnp.float32),
        ),
        mesh=mesh,
        scratch_types=[
            pltpu.VMEM((n_chunks, CHUNK), jnp.int32),
            pltpu.VMEM((n_chunks, CHUNK), jnp.int32),
            pltpu.VMEM((K, CHUNK, d), jnp.float32),
            pltpu.VMEM((K, CHUNK, d), jnp.float32),
            pltpu.SemaphoreType.DMA,
            pltpu.SemaphoreType.DMA,
            pltpu.SemaphoreType.DMA,
            pltpu.SemaphoreType.DMA,
        ],
        compiler_params=pltpu.CompilerParams(use_tc_tiling_on_sc=False),
    )
    def k(src_tab, tgt_tab, sidx_hbm, tidx_hbm, src_out, tgt_out,
          sidx_v, tidx_v, sbuf, tbuf, g0, g1, w0, w1):
        wid = lax.axis_index("s") * NC + lax.axis_index("c")
        base = wid * n_w
        pltpu.sync_copy(sidx_hbm.at[wid], sidx_v)
        pltpu.sync_copy(tidx_hbm.at[wid], tidx_v)

        def fire_gathers(tab, idx_v, buf, sem, p):
            for j in range(K):
                pltpu.async_copy(tab.at[idx_v.at[p * K + j]], buf.at[j], sem)

        def drain_gathers(tab, idx_v, buf, sem, p):
            for j in range(K):
                pltpu.make_async_copy(
                    tab.at[idx_v.at[p * K + j]], buf.at[j], sem).wait()

        def fire_wb(buf, out, sem, p):
            for j in range(K):
                pltpu.async_copy(
                    buf.at[j],
                    out.at[pl.ds(base + (p * K + j) * CHUNK, CHUNK)], sem)

        def drain_wb(buf, out, sem, p):
            for j in range(K):
                pltpu.make_async_copy(
                    buf.at[j],
                    out.at[pl.ds(base + (p * K + j) * CHUNK, CHUNK)],
                    sem).wait()

        def pair(p, first):
            if not first:
                drain_wb(sbuf, src_out, w0, p - 1)
            fire_gathers(src_tab, sidx_v, sbuf, g0, p)
            if not first:
                drain_wb(tbuf, tgt_out, w1, p - 1)
            fire_gathers(tgt_tab, tidx_v, tbuf, g1, p)
            drain_gathers(src_tab, sidx_v, sbuf, g0, p)
            fire_wb(sbuf, src_out, w0, p)
            drain_gathers(tgt_tab, tidx_v, tbuf, g1, p)
            fire_wb(tbuf, tgt_out, w1, p)

        pair(0, True)

        def body(p, carry):
            pair(p, False)
            return carry

        lax.fori_loop(1, n_pairs, body, 0)
        drain_wb(sbuf, src_out, w0, n_pairs - 1)
        drain_wb(tbuf, tgt_out, w1, n_pairs - 1)

    return k(src_table, tgt_table, src_idx, tgt_idx)


def kernel(src_table, tgt_table, src_indices, tgt_indices):
    b, l = src_indices.shape
    d = src_table.shape[1]
    n = b * l
    n_w = n // NW
    n_chunks = n_w // CHUNK
    sidx = src_indices.astype(jnp.int32).reshape(NW, n_chunks, CHUNK)
    tidx = tgt_indices.astype(jnp.int32).reshape(NW, n_chunks, CHUNK)
    src_flat, tgt_flat = _dual_gather(src_table, tgt_table, sidx, tidx, n, d)
    return (src_flat.reshape(b, l, d), tgt_flat.reshape(b, l, d))
